# Initial kernel scaffold; baseline (speedup 1.0000x reference)
import jax
import jax.numpy as jnp
from jax.experimental import pallas as pl


def kernel(x, adj, W1, b1, Wb, bb, W2, b2, bn_g, bn_b, fc1_W, fc1_b, bn1_g, bn1_b, fc2_W, fc2_b, bn2_g, bn2_b, fc3_W, fc3_b):
    raise NotImplementedError("write your pallas kernel here")


# speedup vs baseline: 1.1699x; 1.1699x over previous
import jax
import jax.numpy as jnp
from jax.experimental import pallas as pl
from jax.experimental.pallas import tpu as pltpu

_BN_EPS = 1e-5


def _gcn_body(x_ref, adj_ref, W1_ref, b1_ref, Wb_ref, bb_ref, W2_ref, b2_ref,
              g_ref, beta_ref, out_ref):
    adj = adj_ref[...]
    inv = 1.0 / (1.0 + _BN_EPS) ** 0.5
    gcol = g_ref[...] * inv
    bcol = beta_ref[...]

    def layer(h, W, b):
        hp = jnp.dot(h, W, preferred_element_type=jnp.float32)
        t = jnp.dot(adj, hp, preferred_element_type=jnp.float32) + hp + b
        n = jnp.sqrt(jnp.sum(t * t, axis=1, keepdims=True))
        return t / jnp.maximum(n, 1e-12)

    h = layer(x_ref[0], W1_ref[...], b1_ref[...])
    h = jnp.maximum(h, 0.0) * gcol + bcol
    h = layer(h, Wb_ref[...], bb_ref[...])
    h = jnp.maximum(h, 0.0) * gcol + bcol
    h = layer(h, W2_ref[...], b2_ref[...])
    out_ref[0] = h


def _head_body(hf_ref, fc1W_ref, fc1b_ref, g1_ref, be1_ref, fc2W_ref,
               fc2b_ref, g2_ref, be2_ref, fc3W_ref, fc3b_ref, out_ref):
    inv = 1.0 / (1.0 + _BN_EPS) ** 0.5
    z = jnp.dot(hf_ref[...], fc1W_ref[...],
                preferred_element_type=jnp.float32) + fc1b_ref[...]
    z = jnp.maximum(z, 0.0) * (g1_ref[...] * inv) + be1_ref[...]
    z = jnp.dot(z, fc2W_ref[...],
                preferred_element_type=jnp.float32) + fc2b_ref[...]
    z = jnp.maximum(z, 0.0) * (g2_ref[...] * inv) + be2_ref[...]
    out_ref[...] = jnp.dot(z, fc3W_ref[...],
                           preferred_element_type=jnp.float32) + fc3b_ref[...]


def kernel(x, adj, W1, b1, Wb, bb, W2, b2, bn_g, bn_b, fc1_W, fc1_b, bn1_g,
           bn1_b, fc2_W, fc2_b, bn2_g, bn2_b, fc3_W, fc3_b):
    B, N, F = x.shape
    H = W1.shape[1]
    E = W2.shape[1]
    NH = fc1_W.shape[1]
    L = fc3_W.shape[1]

    rep = lambda shape: pl.BlockSpec(shape, lambda b: (0,) * len(shape))

    h = pl.pallas_call(
        _gcn_body,
        grid=(B,),
        in_specs=[
            pl.BlockSpec((1, N, F), lambda b: (b, 0, 0)),
            rep((N, N)),
            rep((F, H)), rep((1, H)),
            rep((H, H)), rep((1, H)),
            rep((H, E)), rep((1, E)),
            rep((N, 1)), rep((N, 1)),
        ],
        out_specs=pl.BlockSpec((1, N, E), lambda b: (b, 0, 0)),
        out_shape=jax.ShapeDtypeStruct((B, N, E), jnp.float32),
        compiler_params=pltpu.CompilerParams(
            dimension_semantics=("arbitrary",)),
    )(x, adj, W1, b1.reshape(1, H), Wb, bb.reshape(1, H), W2,
      b2.reshape(1, E), bn_g.reshape(N, 1), bn_b.reshape(N, 1))

    ypred = pl.pallas_call(
        _head_body,
        out_shape=jax.ShapeDtypeStruct((B, L), jnp.float32),
    )(h.reshape(B, N * E), fc1_W, fc1_b.reshape(1, NH), bn1_g.reshape(1, NH),
      bn1_b.reshape(1, NH), fc2_W, fc2_b.reshape(1, NH),
      bn2_g.reshape(1, NH), bn2_b.reshape(1, NH), fc3_W, fc3_b.reshape(1, L))

    return (ypred, h)


# speedup vs baseline: 1.3012x; 1.1122x over previous
import jax
import jax.numpy as jnp
from jax.experimental import pallas as pl
from jax.experimental.pallas import tpu as pltpu

_BN_EPS = 1e-5


_BB = 4


def _gcn_body(x_ref, adj_ref, W1_ref, b1_ref, Wb_ref, bb_ref, W2_ref, b2_ref,
              g_ref, beta_ref, out_ref):
    adj = adj_ref[...]
    inv = 1.0 / (1.0 + _BN_EPS) ** 0.5
    gcol = g_ref[...] * inv
    bcol = beta_ref[...]

    def layer(h, W, b):
        hp = jnp.dot(h, W, preferred_element_type=jnp.float32)
        t = jnp.dot(adj, hp, preferred_element_type=jnp.float32) + hp + b
        n = jnp.sqrt(jnp.sum(t * t, axis=1, keepdims=True))
        return t / jnp.maximum(n, 1e-12)

    for i in range(_BB):
        h = layer(x_ref[i], W1_ref[...], b1_ref[...])
        h = jnp.maximum(h, 0.0) * gcol + bcol
        h = layer(h, Wb_ref[...], bb_ref[...])
        h = jnp.maximum(h, 0.0) * gcol + bcol
        h = layer(h, W2_ref[...], b2_ref[...])
        out_ref[i] = h


def _head_body(hf_ref, fc1W_ref, fc1b_ref, g1_ref, be1_ref, fc2W_ref,
               fc2b_ref, g2_ref, be2_ref, fc3W_ref, fc3b_ref, out_ref):
    inv = 1.0 / (1.0 + _BN_EPS) ** 0.5
    z = jnp.dot(hf_ref[...], fc1W_ref[...],
                preferred_element_type=jnp.float32) + fc1b_ref[...]
    z = jnp.maximum(z, 0.0) * (g1_ref[...] * inv) + be1_ref[...]
    z = jnp.dot(z, fc2W_ref[...],
                preferred_element_type=jnp.float32) + fc2b_ref[...]
    z = jnp.maximum(z, 0.0) * (g2_ref[...] * inv) + be2_ref[...]
    out_ref[...] = jnp.dot(z, fc3W_ref[...],
                           preferred_element_type=jnp.float32) + fc3b_ref[...]


def kernel(x, adj, W1, b1, Wb, bb, W2, b2, bn_g, bn_b, fc1_W, fc1_b, bn1_g,
           bn1_b, fc2_W, fc2_b, bn2_g, bn2_b, fc3_W, fc3_b):
    B, N, F = x.shape
    H = W1.shape[1]
    E = W2.shape[1]
    NH = fc1_W.shape[1]
    L = fc3_W.shape[1]

    rep = lambda shape: pl.BlockSpec(shape, lambda b: (0,) * len(shape))

    h = pl.pallas_call(
        _gcn_body,
        grid=(B // _BB,),
        in_specs=[
            pl.BlockSpec((_BB, N, F), lambda b: (b, 0, 0)),
            rep((N, N)),
            rep((F, H)), rep((1, H)),
            rep((H, H)), rep((1, H)),
            rep((H, E)), rep((1, E)),
            rep((N, 1)), rep((N, 1)),
        ],
        out_specs=pl.BlockSpec((_BB, N, E), lambda b: (b, 0, 0)),
        out_shape=jax.ShapeDtypeStruct((B, N, E), jnp.float32),
        compiler_params=pltpu.CompilerParams(
            dimension_semantics=("arbitrary",)),
    )(x, adj, W1, b1.reshape(1, H), Wb, bb.reshape(1, H), W2,
      b2.reshape(1, E), bn_g.reshape(N, 1), bn_b.reshape(N, 1))

    ypred = pl.pallas_call(
        _head_body,
        out_shape=jax.ShapeDtypeStruct((B, L), jnp.float32),
    )(h.reshape(B, N * E), fc1_W, fc1_b.reshape(1, NH), bn1_g.reshape(1, NH),
      bn1_b.reshape(1, NH), fc2_W, fc2_b.reshape(1, NH),
      bn2_g.reshape(1, NH), bn2_b.reshape(1, NH), fc3_W, fc3_b.reshape(1, L))

    return (ypred, h)
